# baseline (speedup 1.0000x reference)
import functools

import jax
import jax.numpy as jnp
from jax import lax
from jax.experimental import pallas as pl
from jax.experimental.pallas import tpu as pltpu
from jax.experimental.pallas import tpu_sc as plsc

VOCAB = 1000000
SEQ = 200
BATCH = 16384
EMB = 3
FEAT = SEQ * EMB
TOTAL = BATCH * SEQ

NC = 2
NS = 16
NW = NC * NS
PER_W = TOTAL // NW
CHUNK = 2048
NCHUNK = PER_W // CHUNK
KSUB = 8
SUB3 = CHUNK * 3 // KSUB


@functools.cache
def _make_gather():
  mesh = plsc.VectorSubcoreMesh(
      core_axis_name="c", subcore_axis_name="s", num_cores=NC, num_subcores=NS
  )

  @functools.partial(
      pl.kernel,
      mesh=mesh,
      out_type=jax.ShapeDtypeStruct((TOTAL * 3,), jnp.float32),
      scratch_types=[
          pltpu.VMEM((CHUNK,), jnp.int32),
          pltpu.VMEM((CHUNK * 3,), jnp.int32),
          pltpu.VMEM((CHUNK * 3,), jnp.float32),
          pltpu.SemaphoreType.DMA,
      ],
      compiler_params=pltpu.CompilerParams(
          use_tc_tiling_on_sc=False, needs_layout_passes=False
      ),
  )
  def gather_kernel(x_hbm, tab3_hbm, out_hbm, idx_v, idx3_v, val_v, sem):
    wid = lax.axis_index("s") * NC + lax.axis_index("c")
    base = wid * PER_W
    iota = lax.iota(jnp.int32, 16)

    def body(j, _):
      o = base + j * CHUNK
      pltpu.sync_copy(x_hbm.at[pl.ds(o, CHUNK)], idx_v)

      def grp(g, _):
        v = idx_v[pl.ds(g * 16, 16)]
        w = v * 3
        pos = (g * 48) + iota * 3
        for d in range(EMB):
          plsc.store_scatter(idx3_v, [pos + d], w + d)
        return 0

      lax.fori_loop(0, CHUNK // 16, grp, 0)
      cps = []
      for i in range(KSUB):
        cps.append(
            pltpu.async_copy(
                tab3_hbm.at[idx3_v.at[pl.ds(i * SUB3, SUB3)]],
                val_v.at[pl.ds(i * SUB3, SUB3)],
                sem,
            )
        )
      for cp in cps:
        cp.wait()
      pltpu.sync_copy(val_v, out_hbm.at[pl.ds(o * 3, CHUNK * 3)])
      return 0

    lax.fori_loop(0, NCHUNK, body, 0)

  return gather_kernel


BB = 1024


def _mlp_body(g_ref, w1_ref, b1_ref, w2_ref, b2_ref, w3_ref, b3_ref, o_ref):
  h = g_ref[...]
  h = jnp.dot(h, w1_ref[...], preferred_element_type=jnp.float32) + b1_ref[...]
  h = jnp.maximum(h, 0.0)
  h = jnp.dot(h, w2_ref[...], preferred_element_type=jnp.float32) + b2_ref[...]
  h = jnp.maximum(h, 0.0)
  z = jnp.dot(h, w3_ref[...], preferred_element_type=jnp.float32) + b3_ref[...]
  o_ref[...] = 1.0 / (1.0 + jnp.exp(-z))


def _mlp(g, w1t, b1, w2t, b2, w3t, b3):
  grid = BATCH // BB
  full = lambda shape: pl.BlockSpec(shape, lambda i: (0, 0))
  return pl.pallas_call(
      _mlp_body,
      grid=(grid,),
      in_specs=[
          pl.BlockSpec((BB, FEAT), lambda i: (i, 0)),
          full((FEAT, 10)),
          full((1, 10)),
          full((10, 5)),
          full((1, 5)),
          full((5, 3)),
          full((1, 3)),
      ],
      out_specs=pl.BlockSpec((BB, 3), lambda i: (i, 0)),
      out_shape=jax.ShapeDtypeStruct((BATCH, 3), jnp.float32),
  )(g, w1t, b1, w2t, b2, w3t, b3)


@jax.jit
def kernel(x, emb, W1, b1, W2, b2, W3, b3):
  x_flat = x.astype(jnp.int32).reshape(TOTAL)
  tab3 = emb.reshape(VOCAB * EMB)
  g = _make_gather()(x_flat, tab3).reshape(BATCH, FEAT)
  return _mlp(
      g,
      W1.T,
      b1.reshape(1, 10),
      W2.T,
      b2.reshape(1, 5),
      W3.T,
      b3.reshape(1, 3),
  )
